# Initial kernel scaffold; baseline (speedup 1.0000x reference)
import jax
import jax.numpy as jnp
from jax.experimental import pallas as pl


def kernel(x_hwc, weight):
    raise NotImplementedError("write your pallas kernel here")


# speedup vs baseline: 1.0992x; 1.0992x over previous
import functools

import jax
import jax.numpy as jnp
from jax.experimental import pallas as pl
from jax.experimental.pallas import tpu as pltpu


def _unpatch_kernel(x_ref, w_ref, o_ref, *, k, w_pix, p, n_cols):
    res = jnp.dot(x_ref[...], w_ref[...], preferred_element_type=jnp.float32)
    for i in range(p):
        sl = res[:, i * n_cols:(i + 1) * n_cols]
        o_ref[:, i, :, :] = sl.reshape(k, w_pix, n_cols).astype(o_ref.dtype)


def kernel(x_hwc: jax.Array, weight: jax.Array) -> jax.Array:
    B, H, W, Cin = x_hwc.shape
    Cin_w, Cout, p, p2 = weight.shape
    assert Cin == Cin_w and p == p2

    bh = B * H
    n_cols = p * Cout

    k = max(1, min(bh, 2048 // max(W, 1)))
    while bh % k:
        k -= 1
    if (bh // k) % 2 and k > 1:
        k //= 2

    grid_m = bh // k

    x2 = x_hwc.reshape(bh * W, Cin)
    w2 = jnp.transpose(weight, (0, 2, 3, 1)).reshape(Cin, p * p * Cout)

    cost = pl.CostEstimate(
        flops=2 * bh * W * Cin * p * p * Cout,
        transcendentals=0,
        bytes_accessed=4 * (bh * W * Cin + Cin * p * p * Cout + bh * W * p * p * Cout),
    )

    kfn = functools.partial(_unpatch_kernel, k=k, w_pix=W, p=p, n_cols=n_cols)
    y = pl.pallas_call(
        kfn,
        out_shape=jax.ShapeDtypeStruct((bh, p, W, n_cols), x_hwc.dtype),
        grid_spec=pl.GridSpec(
            grid=(grid_m,),
            in_specs=[
                pl.BlockSpec((k * W, Cin), lambda m: (m, 0)),
                pl.BlockSpec((Cin, p * p * Cout), lambda m: (0, 0)),
            ],
            out_specs=pl.BlockSpec((k, p, W, n_cols), lambda m: (m, 0, 0, 0)),
        ),
        compiler_params=pltpu.CompilerParams(
            dimension_semantics=("parallel",),
        ),
        cost_estimate=cost,
    )(x2, w2)

    return y.reshape(B, H * p, W * p, Cout)


# speedup vs baseline: 1.1026x; 1.0031x over previous
import functools

import jax
import jax.numpy as jnp
from jax.experimental import pallas as pl
from jax.experimental.pallas import tpu as pltpu

_LANE = 128


def _unpatch_kernel(x_ref, w_ref, o_ref, *, k, w_pix, p, n_cols):
    x = x_ref[...]
    for q in range(p // 2):
        wq = w_ref[:, 2 * q * _LANE:(2 * q + 2) * _LANE]
        rq = jnp.dot(x, wq, preferred_element_type=jnp.float32)
        lo = rq[:, :n_cols].reshape(k, w_pix, n_cols)
        hi = rq[:, _LANE:_LANE + n_cols].reshape(k, w_pix, n_cols)
        o_ref[:, 2 * q, :, :] = lo.astype(o_ref.dtype)
        o_ref[:, 2 * q + 1, :, :] = hi.astype(o_ref.dtype)


def kernel(x_hwc: jax.Array, weight: jax.Array) -> jax.Array:
    B, H, W, Cin = x_hwc.shape
    Cin_w, Cout, p, p2 = weight.shape
    assert Cin == Cin_w and p == p2

    bh = B * H
    n_cols = p * Cout

    k = max(1, min(bh, 2048 // max(W, 1)))
    while bh % k:
        k -= 1
    if (bh // k) % 2 and k > 1:
        k //= 2

    grid_m = bh // k

    x2 = x_hwc.reshape(bh * W, Cin)
    w3 = jnp.transpose(weight, (0, 2, 3, 1)).reshape(Cin, p, n_cols)
    w_pad = jnp.pad(w3, ((0, 0), (0, 0), (0, _LANE - n_cols)))
    w_pad = w_pad.reshape(Cin, p * _LANE)

    cost = pl.CostEstimate(
        flops=2 * bh * W * Cin * p * p * Cout,
        transcendentals=0,
        bytes_accessed=4 * (bh * W * Cin + Cin * p * _LANE + bh * W * p * p * Cout),
    )

    kfn = functools.partial(_unpatch_kernel, k=k, w_pix=W, p=p, n_cols=n_cols)
    y = pl.pallas_call(
        kfn,
        out_shape=jax.ShapeDtypeStruct((bh, p, W, n_cols), x_hwc.dtype),
        grid_spec=pl.GridSpec(
            grid=(grid_m,),
            in_specs=[
                pl.BlockSpec((k * W, Cin), lambda m: (m, 0)),
                pl.BlockSpec((Cin, p * _LANE), lambda m: (0, 0)),
            ],
            out_specs=pl.BlockSpec((k, p, W, n_cols), lambda m: (m, 0, 0, 0)),
        ),
        compiler_params=pltpu.CompilerParams(
            dimension_semantics=("parallel",),
        ),
        cost_estimate=cost,
    )(x2, w_pad)

    return y.reshape(B, H * p, W * p, Cout)


# speedup vs baseline: 1.1055x; 1.0026x over previous
import functools

import jax
import jax.numpy as jnp
from jax.experimental import pallas as pl
from jax.experimental.pallas import tpu as pltpu

_LANE = 128


def _unpatch_kernel(x_ref, w_ref, o_ref, *, k, w_pix, p, n_cols):
    x = x_ref[...]
    for q in range(p // 2):
        wq = w_ref[:, 2 * q * _LANE:(2 * q + 2) * _LANE]
        rq = jnp.dot(x, wq, preferred_element_type=jnp.float32)
        lo = rq[:, :n_cols].reshape(k, w_pix, n_cols)
        hi = rq[:, _LANE:_LANE + n_cols].reshape(k, w_pix, n_cols)
        o_ref[:, 2 * q, :, :] = lo.astype(o_ref.dtype)
        o_ref[:, 2 * q + 1, :, :] = hi.astype(o_ref.dtype)


def kernel(x_hwc: jax.Array, weight: jax.Array) -> jax.Array:
    B, H, W, Cin = x_hwc.shape
    Cin_w, Cout, p, p2 = weight.shape
    assert Cin == Cin_w and p == p2

    bh = B * H
    n_cols = p * Cout

    k = max(1, min(bh, 4096 // max(W, 1)))
    while bh % k:
        k -= 1
    if (bh // k) % 2 and k > 1:
        k //= 2

    grid_m = bh // k

    x2 = x_hwc.reshape(bh * W, Cin)
    w3 = jnp.transpose(weight, (0, 2, 3, 1)).reshape(Cin, p, n_cols)
    w_pad = jnp.pad(w3, ((0, 0), (0, 0), (0, _LANE - n_cols)))
    w_pad = w_pad.reshape(Cin, p * _LANE)

    cost = pl.CostEstimate(
        flops=2 * bh * W * Cin * p * p * Cout,
        transcendentals=0,
        bytes_accessed=4 * (bh * W * Cin + Cin * p * _LANE + bh * W * p * p * Cout),
    )

    kfn = functools.partial(_unpatch_kernel, k=k, w_pix=W, p=p, n_cols=n_cols)
    y = pl.pallas_call(
        kfn,
        out_shape=jax.ShapeDtypeStruct((bh, p, W, n_cols), x_hwc.dtype),
        grid_spec=pl.GridSpec(
            grid=(grid_m,),
            in_specs=[
                pl.BlockSpec((k * W, Cin), lambda m: (m, 0)),
                pl.BlockSpec((Cin, p * _LANE), lambda m: (0, 0)),
            ],
            out_specs=pl.BlockSpec((k, p, W, n_cols), lambda m: (m, 0, 0, 0)),
        ),
        compiler_params=pltpu.CompilerParams(
            dimension_semantics=("parallel",),
        ),
        cost_estimate=cost,
    )(x2, w_pad)

    return y.reshape(B, H * p, W * p, Cout)
